# baseline (speedup 1.0000x reference)
import functools

import jax
import jax.numpy as jnp
from jax import lax
from jax.experimental import pallas as pl
from jax.experimental.pallas import tpu as pltpu
from jax.experimental.pallas import tpu_sc as plsc

BATCH = 16384
D = 32
BN_EPS = 1e-5

NC = 2
NS = 16
NW = NC * NS
BPW = BATCH // NW
CH = 128
NCH = BPW // CH

_IDX_ROWS = BATCH // CH


def _sc_gather(uidx2, iidx2, ug_t, ig_t, um_t, im_t):
    mesh = plsc.VectorSubcoreMesh(core_axis_name="c", subcore_axis_name="s")

    @functools.partial(
        pl.kernel,
        mesh=mesh,
        compiler_params=pltpu.CompilerParams(use_tc_tiling_on_sc=False),
        out_type=[jax.ShapeDtypeStruct((BATCH, D), jnp.float32)] * 4,
        scratch_types=[
            pltpu.VMEM((NCH, CH), jnp.int32),
            pltpu.VMEM((NCH, CH), jnp.int32),
            pltpu.VMEM((BPW, D), jnp.float32),
            pltpu.VMEM((BPW, D), jnp.float32),
            pltpu.VMEM((BPW, D), jnp.float32),
            pltpu.VMEM((BPW, D), jnp.float32),
            pltpu.SemaphoreType.DMA,
            pltpu.SemaphoreType.DMA,
            pltpu.SemaphoreType.DMA,
            pltpu.SemaphoreType.DMA,
        ],
    )
    def k(uidx_hbm, iidx_hbm, ugt, igt, umt, imt,
          out_ug, out_ig, out_um, out_im,
          uix, iix, ugv, igv, umv, imv, s0, s1, s2, s3):
        wid = lax.axis_index("s") * NC + lax.axis_index("c")
        base = wid * BPW
        row0 = wid * NCH
        pltpu.sync_copy(uidx_hbm.at[pl.ds(row0, NCH)], uix)
        pltpu.sync_copy(iidx_hbm.at[pl.ds(row0, NCH)], iix)
        copies = []
        for j in range(NCH):
            dst = pl.ds(j * CH, CH)
            copies.append(pltpu.async_copy(ugt.at[uix.at[j]], ugv.at[dst], s0))
            copies.append(pltpu.async_copy(igt.at[iix.at[j]], igv.at[dst], s1))
            copies.append(pltpu.async_copy(umt.at[uix.at[j]], umv.at[dst], s2))
            copies.append(pltpu.async_copy(imt.at[iix.at[j]], imv.at[dst], s3))
        for c in copies:
            c.wait()
        out = pl.ds(base, BPW)
        pltpu.sync_copy(ugv, out_ug.at[out])
        pltpu.sync_copy(igv, out_ig.at[out])
        pltpu.sync_copy(umv, out_um.at[out])
        pltpu.sync_copy(imv, out_im.at[out])

    return k(uidx2, iidx2, ug_t, ig_t, um_t, im_t)


def _tc_body(ug_ref, ig_ref, um_ref, im_ref,
             w0u, w0i, b0, w1, b1, w2, b2, w3, b3, wpg, wph, bp,
             out_ref):
    f32 = jnp.float32
    hi = jax.lax.Precision.HIGHEST
    ug = ug_ref[...]
    ig = ig_ref[...]
    nu = jnp.sqrt(jnp.sum(ug * ug, axis=1, keepdims=True))
    ni = jnp.sqrt(jnp.sum(ig * ig, axis=1, keepdims=True))
    gmf = (ug / jnp.maximum(nu, 1e-12)) * (ig / jnp.maximum(ni, 1e-12))
    h = (jnp.dot(um_ref[...], w0u[...], preferred_element_type=f32, precision=hi)
         + jnp.dot(im_ref[...], w0i[...], preferred_element_type=f32, precision=hi)
         + b0[...])
    h = jnp.maximum(h, 0.0)
    for w, b in ((w1, b1), (w2, b2), (w3, b3)):
        h = jnp.dot(h, w[...], preferred_element_type=f32, precision=hi) + b[...]
        h = jnp.maximum(h, 0.0)
    pred = (jnp.dot(gmf, wpg[...], preferred_element_type=f32, precision=hi)
            + jnp.dot(h, wph[...], preferred_element_type=f32, precision=hi)
            + bp[...])
    out_ref[...] = jax.nn.sigmoid(pred)


def kernel(user_indices, item_indices, user_emb_gmf, item_emb_gmf,
           user_emb_mlp, item_emb_mlp,
           W0, b0, gamma0, beta0, W1, b1, gamma1, beta1,
           W2, b2, gamma2, beta2, W3, b3, gamma3, beta3,
           Wp, bp):
    uidx2 = user_indices.astype(jnp.int32).reshape(_IDX_ROWS, CH)
    iidx2 = item_indices.astype(jnp.int32).reshape(_IDX_ROWS, CH)

    ug, ig, um, im = _sc_gather(uidx2, iidx2, user_emb_gmf, item_emb_gmf,
                                user_emb_mlp, item_emb_mlp)

    bn = 1.0 / jnp.sqrt(1.0 + BN_EPS)
    def fold(W, b, g, be):
        s = g * bn
        return W * s[None, :], (b * s + be)[None, :]
    W0f, b0f = fold(W0, b0, gamma0, beta0)
    W1f, b1f = fold(W1, b1, gamma1, beta1)
    W2f, b2f = fold(W2, b2, gamma2, beta2)
    W3f, b3f = fold(W3, b3, gamma3, beta3)
    w0u, w0i = W0f[:D], W0f[D:]
    wpg, wph = Wp[:D], Wp[D:]
    bp2 = bp[None, :]

    BB = 2048
    grid = (BATCH // BB,)
    emb_spec = pl.BlockSpec((BB, D), lambda i: (i, 0))
    def w_spec(a):
        return pl.BlockSpec(a.shape, lambda i: (0,) * a.ndim)
    weights = (w0u, w0i, b0f, W1f, b1f, W2f, b2f, W3f, b3f, wpg, wph, bp2)

    out = pl.pallas_call(
        _tc_body,
        grid=grid,
        in_specs=[emb_spec] * 4 + [w_spec(a) for a in weights],
        out_specs=pl.BlockSpec((BB, 1), lambda i: (i, 0)),
        out_shape=jax.ShapeDtypeStruct((BATCH, 1), jnp.float32),
    )(ug, ig, um, im, *weights)
    return out
